# baseline (speedup 1.0000x reference)
import functools

import jax
import jax.numpy as jnp
from jax import lax
from jax.experimental import pallas as pl
from jax.experimental.pallas import tpu as pltpu
from jax.experimental.pallas import tpu_sc as plsc

NC = 2
NS = 16
NW = NC * NS
G = 8


def _emit_bits(dst, src, dst0, src0, n_bits, rem):
    off = jnp.int32(0)
    for b in reversed(range(n_bits)):
        s = 1 << b
        pred = ((rem >> b) & 1) == 1
        o = off

        @pl.when(pred)
        def _():
            pltpu.sync_copy(src.at[pl.ds(src0 + o, s)], dst.at[pl.ds(dst0 + o, s)])

        off = off + jnp.where(pred, jnp.int32(s), jnp.int32(0))


def _ring_dma(dst, src, a, src0, size, m, n_bits, src_is_ring):
    t = m - a

    @pl.when(t >= size)
    def _():
        pltpu.sync_copy(src.at[pl.ds(src0, size)], dst.at[pl.ds(a, size)])

    @pl.when(t < size)
    def _():
        _emit_bits(dst, src, a, src0, n_bits, t)
        src_b = jnp.int32(0) if src_is_ring else src0 + t
        _emit_bits(dst, src, jnp.int32(0), src_b, n_bits, size - t)


def kernel(storage, vals, next_ptr):
    cap, dim = storage.shape
    batch = vals.shape[0]
    next_ptr = jnp.asarray(next_ptr, jnp.int32)

    m = cap // G
    sg = batch // G
    s_per_w = sg // NW
    cg = m - sg
    c_per_w = -(-cg // NW)
    sbits = s_per_w.bit_length()
    cbits = c_per_w.bit_length()

    storage3 = storage.reshape(m, G, dim)
    vals3 = vals.reshape(sg, G, dim)
    base_g = jnp.full((16,), next_ptr // G, dtype=jnp.float32)

    mesh = plsc.VectorSubcoreMesh(core_axis_name="c", subcore_axis_name="s")

    @functools.partial(
        pl.kernel,
        mesh=mesh,
        out_type=jax.ShapeDtypeStruct((m, G, dim), jnp.float32),
        scratch_types=[pltpu.VMEM((16,), jnp.float32)],
        compiler_params=pltpu.CompilerParams(needs_layout_passes=False),
    )
    def sc_fifo(storage_hbm, vals_hbm, base_hbm, out_hbm, base_vmem):
        wid = lax.axis_index("s") * NC + lax.axis_index("c")
        pltpu.sync_copy(base_hbm, base_vmem)
        bg = jnp.max(base_vmem[...]).astype(jnp.int32)

        d0 = wid * s_per_w
        a_s = lax.rem(bg + d0, m)
        _ring_dma(out_hbm, vals_hbm, a_s, d0, s_per_w, m, sbits, False)

        cstart = jnp.minimum(wid * c_per_w, cg - c_per_w)
        a_c = lax.rem(bg + sg + cstart, m)
        _ring_dma(out_hbm, storage_hbm, a_c, a_c, c_per_w, m, cbits, True)

    new_storage3 = sc_fifo(storage3, vals3, base_g)
    new_storage = new_storage3.reshape(cap, dim)
    new_ptr = (next_ptr + batch) % cap
    return new_storage, new_ptr.astype(jnp.int32)


# speedup vs baseline: 20.0138x; 20.0138x over previous
import functools

import jax
import jax.numpy as jnp
from jax import lax
from jax.experimental import pallas as pl
from jax.experimental.pallas import tpu as pltpu
from jax.experimental.pallas import tpu_sc as plsc

NC = 2
NS = 16
NW = NC * NS
G = 8


def _emit_bits(dst, src, dst0, src0, n_bits, rem):
    off = jnp.int32(0)
    for b in reversed(range(n_bits)):
        s = 1 << b
        pred = ((rem >> b) & 1) == 1
        o = off

        @pl.when(pred)
        def _():
            pltpu.sync_copy(src.at[pl.ds(src0 + o, s)], dst.at[pl.ds(dst0 + o, s)])

        off = off + jnp.where(pred, jnp.int32(s), jnp.int32(0))


def _ring_scatter(dst, src, a, size, m, n_bits):
    t = m - a

    @pl.when(t >= size)
    def _():
        pltpu.sync_copy(src.at[pl.ds(0, size)], dst.at[pl.ds(a, size)])

    @pl.when(t < size)
    def _():
        _emit_bits(dst, src, a, jnp.int32(0), n_bits, t)
        _emit_bits(dst, src, jnp.int32(0), t, n_bits, size - t)


def kernel(storage, vals, next_ptr):
    cap, dim = storage.shape
    batch = vals.shape[0]
    next_ptr = jnp.asarray(next_ptr, jnp.int32)

    m = cap // G
    sg = batch // G
    s_per_w = sg // NW
    sbits = s_per_w.bit_length()

    storage3 = storage.reshape(m, G, dim)
    vals3 = vals.reshape(sg, G, dim)
    base_g = jnp.full((16,), next_ptr // G, dtype=jnp.float32)

    mesh = plsc.VectorSubcoreMesh(core_axis_name="c", subcore_axis_name="s")

    @functools.partial(
        pl.kernel,
        mesh=mesh,
        scratch_types=[
            pltpu.VMEM((s_per_w, G, dim), jnp.float32),
            pltpu.VMEM((16,), jnp.float32),
        ],
        compiler_params=pltpu.CompilerParams(needs_layout_passes=False),
    )
    def sc_fifo(out_hbm, vals_hbm, base_hbm, buf, base_vmem):
        wid = lax.axis_index("s") * NC + lax.axis_index("c")
        pltpu.sync_copy(base_hbm, base_vmem)
        bg = jnp.max(base_vmem[...]).astype(jnp.int32)

        d0 = wid * s_per_w
        pltpu.sync_copy(vals_hbm.at[pl.ds(d0, s_per_w)], buf)
        a_s = lax.rem(bg + d0, m)
        _ring_scatter(out_hbm, buf, a_s, s_per_w, m, sbits)

    out_ref = jax.new_ref(storage3)
    sc_fifo(out_ref, vals3, base_g)
    new_storage = out_ref[...].reshape(cap, dim)
    new_ptr = (next_ptr + batch) % cap
    return new_storage, new_ptr.astype(jnp.int32)
